# baseline (speedup 1.0000x reference)
import functools

import jax
import jax.numpy as jnp
from jax import lax
from jax.experimental import pallas as pl
from jax.experimental.pallas import tpu as pltpu

_F32 = jnp.float32
_BF16 = jnp.bfloat16


def _dtc_body(xa_ref, xb_ref, cw_ref, cb_ref,
              w1ih_ref, w1hh_ref, b1_ref, w2ih_ref, b2_ref, whh2_ref,
              out_ref,
              xg1_scr, act_scr, h1_scr, xg2t_scr, hf_scr, hb_scr,
              h_scr, c_scr,
              *, TC, NP, H, L2, K, Cout, G):
    gid = pl.program_id(0)
    ci = pl.program_id(1)
    NC = pl.num_programs(1) - 1
    w1ih = w1ih_ref[...]
    b1 = b1_ref[...]

    def _conv_act(lo_ref, hi_ref):
        xa = lo_ref[0]
        xb = hi_ref[0][:, :, :K - 1]
        HP = TC // 2 * NP
        for hh in range(2):
            if hh == 0:
                xsl = xa[:, :, :TC + K - 1]
            else:
                xsl = jnp.concatenate([xa[:, :, TC:], xb], axis=2)
            xh = jnp.transpose(xsl, (2, 0, 1))
            xf = xh.reshape((TC + K - 1) * NP, xh.shape[2])
            y = jnp.dot(xf[:TC * NP], cw_ref[0],
                        preferred_element_type=_F32)
            for k in range(1, K):
                y = y + jnp.dot(xf[k * NP:(TC + k) * NP], cw_ref[k],
                                preferred_element_type=_F32)
            y3 = y.reshape(TC // 2, 2 * NP, Cout)
            a = jnp.maximum(y3[:, :NP, :], y3[:, NP:, :]).reshape(HP, Cout)
            a = a + cb_ref[...]
            act_scr[pl.ds(hh * HP, HP), :] = jnp.where(a >= 0.0, a, 0.01 * a)

    _conv_act(xa_ref, xb_ref)

    @pl.when(ci == 0)
    def _init():
        h_scr[...] = jnp.zeros_like(h_scr)
        c_scr[...] = jnp.zeros_like(c_scr)
        xg1_scr[...] = jnp.dot(act_scr[...], w1ih,
                               preferred_element_type=_F32) + b1

    w1hh = w1hh_ref[...]
    NCH = 2
    HNP = NP // NCH

    def _cell(r0, h, c):
        g = xg1_scr[pl.ds(r0, HNP), :] + jnp.dot(
            h, w1hh, preferred_element_type=_F32)
        th = jnp.tanh(g[:, :3 * H])
        tg = jnp.tanh(g[:, 3 * H:])
        c = 0.5 * ((c + tg) + (th[:, H:2 * H] * c + th[:, :H] * tg))
        tc = jnp.tanh(c)
        h = (tc + th[:, 2 * H:3 * H] * tc).astype(_BF16)
        h1_scr[pl.ds(r0, HNP), :] = h
        return h, c

    @pl.when(ci > 0)
    def _recur():
        UN = next(u for u in (8, 4, 2, 1) if TC % u == 0)

        def l1_block(tb, carry):
            hs, cs = carry
            base = pl.multiple_of(tb * UN * NP, NP)
            for u in range(UN):
                r0 = base + u * NP
                new = [_cell(r0 + q * HNP, hs[q], cs[q]) for q in range(NCH)]
                hs = tuple(n[0] for n in new)
                cs = tuple(n[1] for n in new)
            xg1_scr[pl.ds(base, UN * NP), :] = jnp.dot(
                act_scr[pl.ds(base, UN * NP), :], w1ih,
                preferred_element_type=_F32) + b1
            return hs, cs

        carry0 = (tuple(h_scr[q * HNP:(q + 1) * HNP, :] for q in range(NCH)),
                  tuple(c_scr[q * HNP:(q + 1) * HNP, :] for q in range(NCH)))
        hs, cs = lax.fori_loop(0, TC // UN, l1_block, carry0)
        for q in range(NCH):
            h_scr[q * HNP:(q + 1) * HNP, :] = hs[q]
            c_scr[q * HNP:(q + 1) * HNP, :] = cs[q]

        g2 = jnp.dot(h1_scr[...], w2ih_ref[...],
                     preferred_element_type=_F32) + b2_ref[...]
        g2t = jnp.transpose(g2.reshape(TC, NP, 8),
                            (0, 2, 1)).reshape(TC * 8, NP)
        xg2t_scr[gid,
                 pl.ds(pl.multiple_of((ci - 1) * TC * 8, 8), TC * 8), :] = g2t

    @pl.when((gid == G - 1) & (ci == NC))
    def _lstm2():
        whh2 = whh2_ref[...]
        fmask = (lax.broadcasted_iota(jnp.int32, (8, 1), 0) & 1) == 0

        def _cell2(gq, s, rb, h2, c2):
            row_f = xg2t_scr[gq, pl.ds(pl.multiple_of(s * 8, 8), 8), :]
            row_b = xg2t_scr[gq, pl.ds(pl.multiple_of(rb * 8, 8), 8), :]
            g = jnp.where(fmask, row_f, row_b) + jnp.concatenate(
                [h2, h2, h2, h2], axis=0) * whh2
            sg = 0.5 * jnp.tanh(g[0:6, :]) + 0.5
            gg = jnp.tanh(g[6:8, :])
            c2 = sg[2:4, :] * c2 + sg[0:2, :] * gg
            h2 = sg[4:6, :] * jnp.tanh(c2)
            hf_scr[gq, pl.ds(s, 1), :] = h2[0:1, :]
            hb_scr[gq, pl.ds(rb, 1), :] = h2[1:2, :]
            return h2, c2

        def l2_step(s, carry):
            rb = L2 - 1 - s
            hs2, cs2 = carry
            new = [_cell2(gq, s, rb, hs2[gq], cs2[gq]) for gq in range(G)]
            return tuple(n[0] for n in new), tuple(n[1] for n in new)

        zero2 = jnp.zeros((2, NP), _F32)
        lax.fori_loop(0, L2, l2_step,
                      ((zero2,) * G, (zero2,) * G), unroll=8)
        for gq in range(G):
            out_ref[pl.ds(gq * NP, NP), :] = jnp.transpose(
                hf_scr[gq] + hb_scr[gq])


def kernel(conv_w, conv_b, l1_w_ih, l1_w_hh, l1_b_ih, l1_b_hh,
           l2f_w_ih, l2f_w_hh, l2f_b_ih, l2f_b_hh,
           l2b_w_ih, l2b_w_hh, l2b_b_ih, l2b_b_hh, x):
    Cout, Cin, K = conv_w.shape
    H = l1_w_hh.shape[1]
    B, _, L = x.shape
    L1 = L - K + 1
    L2 = (L1 - 2) // 2 + 1

    NP = 128
    G = pl.cdiv(B, NP)
    B_pad = G * NP
    TC = next(t for t in (64, 32, 16, 8, 4, 2, 1) if L2 % t == 0)
    NC = L2 // TC
    NBL = pl.cdiv(L, 2 * TC)

    x4 = jnp.pad(x.astype(_F32), ((0, B_pad - B), (0, 0), (0, 0)))
    x4 = x4.reshape(G, NP, Cin, L)

    cw = jnp.transpose(conv_w.astype(_F32), (2, 1, 0))
    cb = conv_b.reshape(1, Cout).astype(_F32)

    perm1 = jnp.concatenate([jnp.arange(0, 2 * H),
                             jnp.arange(3 * H, 4 * H),
                             jnp.arange(2 * H, 3 * H)])
    sc1 = jnp.concatenate([jnp.full((3 * H,), 0.5, _F32),
                           jnp.ones((H,), _F32)])[None, :]
    w1ih = (l1_w_ih[perm1, :].T * sc1).astype(_F32)
    w1hh = (l1_w_hh[perm1, :].T * (0.5 * sc1)).astype(_BF16)
    b1 = ((l1_b_ih + l1_b_hh)[perm1].reshape(1, 4 * H) * sc1).astype(_F32)

    perm2 = jnp.array([0, 1, 3, 2])
    w2f = l2f_w_ih[perm2, :].T.astype(_F32)
    w2b = l2b_w_ih[perm2, :].T.astype(_F32)
    sc2 = jnp.concatenate([jnp.full((6,), 0.5, _F32),
                           jnp.ones((2,), _F32)])
    w2ih = (jnp.stack([w2f, w2b], axis=2).reshape(H, 8)
            * sc2[None, :]).astype(_F32)
    b2f = (l2f_b_ih + l2f_b_hh)[perm2]
    b2b = (l2b_b_ih + l2b_b_hh)[perm2]
    b2 = (jnp.stack([b2f, b2b], axis=1).reshape(1, 8)
          * sc2[None, :]).astype(_F32)
    whh2 = (jnp.stack([l2f_w_hh[perm2, 0], l2b_w_hh[perm2, 0]],
                      axis=1).reshape(8, 1) * sc2[:, None]).astype(_F32)

    weights = (cw, cb, w1ih, w1hh, b1, w2ih, b2, whh2)

    def full_spec(a):
        nd = a.ndim
        return pl.BlockSpec(a.shape, lambda g, ci, nd=nd: (0,) * nd)

    def xspec(off):
        return pl.BlockSpec(
            (1, NP, Cin, 2 * TC),
            lambda g, ci, off=off: (g, 0, 0, jnp.minimum(ci + off, NBL - 1)))

    body = functools.partial(_dtc_body, TC=TC, NP=NP, H=H, L2=L2, K=K,
                             Cout=Cout, G=G)

    out = pl.pallas_call(
        body,
        out_shape=jax.ShapeDtypeStruct((B_pad, L2), _F32),
        grid_spec=pltpu.PrefetchScalarGridSpec(
            num_scalar_prefetch=0,
            grid=(G, NC + 1),
            in_specs=[xspec(0), xspec(1)]
                     + [full_spec(a) for a in weights],
            out_specs=pl.BlockSpec((B_pad, L2), lambda g, ci: (0, 0)),
            scratch_shapes=[
                pltpu.VMEM((TC * NP, 4 * H), _F32),
                pltpu.VMEM((TC * NP, Cout), _F32),
                pltpu.VMEM((TC * NP, H), _BF16),
                pltpu.VMEM((G, L2 * 8, NP), _F32),
                pltpu.VMEM((G, L2, NP), _F32),
                pltpu.VMEM((G, L2, NP), _F32),
                pltpu.VMEM((NP, H), _BF16),
                pltpu.VMEM((NP, H), _F32),
            ]),
        compiler_params=pltpu.CompilerParams(
            dimension_semantics=("arbitrary", "arbitrary"),
            vmem_limit_bytes=64 * 1024 * 1024),
    )(x4, x4, *weights)

    return out[:B][:, None, :]


# speedup vs baseline: 1.0752x; 1.0752x over previous
import functools

import jax
import jax.numpy as jnp
from jax import lax
from jax.experimental import pallas as pl
from jax.experimental.pallas import tpu as pltpu

_F32 = jnp.float32
_BF16 = jnp.bfloat16


def _dtc_body(xa_ref, xb_ref, cw_ref, cb_ref,
              w1ih_ref, w1hh_ref, b1_ref, w2ih_ref, b2_ref, whh2_ref,
              out_ref,
              xg1_scr, act_scr, h1_scr, xg2t_scr, hf_scr, hb_scr,
              h_scr, c_scr,
              *, TC, NP, H, L2, K, Cout, G):
    gid = pl.program_id(0)
    ci = pl.program_id(1)
    NC = pl.num_programs(1) - 1
    w1ih = w1ih_ref[...]
    b1 = b1_ref[...]

    def _conv_act(lo_ref, hi_ref):
        xa = lo_ref[0]
        xb = hi_ref[0][:, :, :K - 1]
        HP = TC // 2 * NP
        for hh in range(2):
            if hh == 0:
                xsl = xa[:, :, :TC + K - 1]
            else:
                xsl = jnp.concatenate([xa[:, :, TC:], xb], axis=2)
            xh = jnp.transpose(xsl, (2, 0, 1))
            xf = xh.reshape((TC + K - 1) * NP, xh.shape[2])
            y = jnp.dot(xf[:TC * NP], cw_ref[0],
                        preferred_element_type=_F32)
            for k in range(1, K):
                y = y + jnp.dot(xf[k * NP:(TC + k) * NP], cw_ref[k],
                                preferred_element_type=_F32)
            y3 = y.reshape(TC // 2, 2 * NP, Cout)
            a = jnp.maximum(y3[:, :NP, :], y3[:, NP:, :]).reshape(HP, Cout)
            a = a + cb_ref[...]
            act_scr[pl.ds(hh * HP, HP), :] = jnp.where(a >= 0.0, a, 0.01 * a)

    _conv_act(xa_ref, xb_ref)

    @pl.when(ci == 0)
    def _init():
        h_scr[...] = jnp.zeros_like(h_scr)
        c_scr[...] = jnp.zeros_like(c_scr)
        xg1_scr[...] = jnp.dot(act_scr[...], w1ih,
                               preferred_element_type=_F32) + b1

    w1hh = w1hh_ref[...]
    NCH = 2
    HNP = NP // NCH

    def _cell(r0, h, c):
        g = xg1_scr[pl.ds(r0, HNP), :] + jnp.dot(
            h, w1hh, preferred_element_type=_F32)
        th = jnp.tanh(g[:, :3 * H])
        tg = jnp.tanh(g[:, 3 * H:])
        c = 0.5 * ((c + tg) + (th[:, H:2 * H] * c + th[:, :H] * tg))
        tc = jnp.tanh(c)
        h = (tc + th[:, 2 * H:3 * H] * tc).astype(_BF16)
        h1_scr[pl.ds(r0, HNP), :] = h
        return h, c

    @pl.when(ci > 0)
    def _recur():
        UN = next(u for u in (8, 4, 2, 1) if TC % u == 0)

        def l1_block(tb, carry):
            hs, cs = carry
            base = pl.multiple_of(tb * UN * NP, NP)
            for u in range(UN):
                r0 = base + u * NP
                new = [_cell(r0 + q * HNP, hs[q], cs[q]) for q in range(NCH)]
                hs = tuple(n[0] for n in new)
                cs = tuple(n[1] for n in new)
                xg1_scr[pl.ds(r0, NP), :] = jnp.dot(
                    act_scr[pl.ds(r0, NP), :], w1ih,
                    preferred_element_type=_F32) + b1
            return hs, cs

        carry0 = (tuple(h_scr[q * HNP:(q + 1) * HNP, :] for q in range(NCH)),
                  tuple(c_scr[q * HNP:(q + 1) * HNP, :] for q in range(NCH)))
        hs, cs = lax.fori_loop(0, TC // UN, l1_block, carry0)
        for q in range(NCH):
            h_scr[q * HNP:(q + 1) * HNP, :] = hs[q]
            c_scr[q * HNP:(q + 1) * HNP, :] = cs[q]

        g2 = jnp.dot(h1_scr[...], w2ih_ref[...],
                     preferred_element_type=_F32) + b2_ref[...]
        g2t = jnp.transpose(g2.reshape(TC, NP, 8),
                            (0, 2, 1)).reshape(TC * 8, NP)
        xg2t_scr[gid,
                 pl.ds(pl.multiple_of((ci - 1) * TC * 8, 8), TC * 8), :] = g2t

    @pl.when((gid == G - 1) & (ci == NC))
    def _lstm2():
        whh2 = whh2_ref[...]
        fmask = (lax.broadcasted_iota(jnp.int32, (8, 1), 0) & 1) == 0

        def _cell2(gq, s, rb, h2, c2):
            row_f = xg2t_scr[gq, pl.ds(pl.multiple_of(s * 8, 8), 8), :]
            row_b = xg2t_scr[gq, pl.ds(pl.multiple_of(rb * 8, 8), 8), :]
            g = jnp.where(fmask, row_f, row_b) + jnp.concatenate(
                [h2, h2, h2, h2], axis=0) * whh2
            sg = 0.5 * jnp.tanh(g[0:6, :]) + 0.5
            gg = jnp.tanh(g[6:8, :])
            c2 = sg[2:4, :] * c2 + sg[0:2, :] * gg
            h2 = sg[4:6, :] * jnp.tanh(c2)
            hf_scr[gq, pl.ds(s, 1), :] = h2[0:1, :]
            hb_scr[gq, pl.ds(rb, 1), :] = h2[1:2, :]
            return h2, c2

        def l2_step(s, carry):
            rb = L2 - 1 - s
            hs2, cs2 = carry
            new = [_cell2(gq, s, rb, hs2[gq], cs2[gq]) for gq in range(G)]
            return tuple(n[0] for n in new), tuple(n[1] for n in new)

        zero2 = jnp.zeros((2, NP), _F32)
        lax.fori_loop(0, L2, l2_step,
                      ((zero2,) * G, (zero2,) * G), unroll=8)
        for gq in range(G):
            out_ref[pl.ds(gq * NP, NP), :] = jnp.transpose(
                hf_scr[gq] + hb_scr[gq])


def kernel(conv_w, conv_b, l1_w_ih, l1_w_hh, l1_b_ih, l1_b_hh,
           l2f_w_ih, l2f_w_hh, l2f_b_ih, l2f_b_hh,
           l2b_w_ih, l2b_w_hh, l2b_b_ih, l2b_b_hh, x):
    Cout, Cin, K = conv_w.shape
    H = l1_w_hh.shape[1]
    B, _, L = x.shape
    L1 = L - K + 1
    L2 = (L1 - 2) // 2 + 1

    NP = 128
    G = pl.cdiv(B, NP)
    B_pad = G * NP
    TC = next(t for t in (64, 32, 16, 8, 4, 2, 1) if L2 % t == 0)
    NC = L2 // TC
    NBL = pl.cdiv(L, 2 * TC)

    x4 = jnp.pad(x.astype(_F32), ((0, B_pad - B), (0, 0), (0, 0)))
    x4 = x4.reshape(G, NP, Cin, L)

    cw = jnp.transpose(conv_w.astype(_F32), (2, 1, 0))
    cb = conv_b.reshape(1, Cout).astype(_F32)

    perm1 = jnp.concatenate([jnp.arange(0, 2 * H),
                             jnp.arange(3 * H, 4 * H),
                             jnp.arange(2 * H, 3 * H)])
    sc1 = jnp.concatenate([jnp.full((3 * H,), 0.5, _F32),
                           jnp.ones((H,), _F32)])[None, :]
    w1ih = (l1_w_ih[perm1, :].T * sc1).astype(_F32)
    w1hh = (l1_w_hh[perm1, :].T * (0.5 * sc1)).astype(_BF16)
    b1 = ((l1_b_ih + l1_b_hh)[perm1].reshape(1, 4 * H) * sc1).astype(_F32)

    perm2 = jnp.array([0, 1, 3, 2])
    w2f = l2f_w_ih[perm2, :].T.astype(_F32)
    w2b = l2b_w_ih[perm2, :].T.astype(_F32)
    sc2 = jnp.concatenate([jnp.full((6,), 0.5, _F32),
                           jnp.ones((2,), _F32)])
    w2ih = (jnp.stack([w2f, w2b], axis=2).reshape(H, 8)
            * sc2[None, :]).astype(_F32)
    b2f = (l2f_b_ih + l2f_b_hh)[perm2]
    b2b = (l2b_b_ih + l2b_b_hh)[perm2]
    b2 = (jnp.stack([b2f, b2b], axis=1).reshape(1, 8)
          * sc2[None, :]).astype(_F32)
    whh2 = (jnp.stack([l2f_w_hh[perm2, 0], l2b_w_hh[perm2, 0]],
                      axis=1).reshape(8, 1) * sc2[:, None]).astype(_F32)

    weights = (cw, cb, w1ih, w1hh, b1, w2ih, b2, whh2)

    def full_spec(a):
        nd = a.ndim
        return pl.BlockSpec(a.shape, lambda g, ci, nd=nd: (0,) * nd)

    def xspec(off):
        return pl.BlockSpec(
            (1, NP, Cin, 2 * TC),
            lambda g, ci, off=off: (g, 0, 0, jnp.minimum(ci + off, NBL - 1)))

    body = functools.partial(_dtc_body, TC=TC, NP=NP, H=H, L2=L2, K=K,
                             Cout=Cout, G=G)

    out = pl.pallas_call(
        body,
        out_shape=jax.ShapeDtypeStruct((B_pad, L2), _F32),
        grid_spec=pltpu.PrefetchScalarGridSpec(
            num_scalar_prefetch=0,
            grid=(G, NC + 1),
            in_specs=[xspec(0), xspec(1)]
                     + [full_spec(a) for a in weights],
            out_specs=pl.BlockSpec((B_pad, L2), lambda g, ci: (0, 0)),
            scratch_shapes=[
                pltpu.VMEM((TC * NP, 4 * H), _F32),
                pltpu.VMEM((TC * NP, Cout), _F32),
                pltpu.VMEM((TC * NP, H), _BF16),
                pltpu.VMEM((G, L2 * 8, NP), _F32),
                pltpu.VMEM((G, L2, NP), _F32),
                pltpu.VMEM((G, L2, NP), _F32),
                pltpu.VMEM((NP, H), _BF16),
                pltpu.VMEM((NP, H), _F32),
            ]),
        compiler_params=pltpu.CompilerParams(
            dimension_semantics=("arbitrary", "arbitrary"),
            vmem_limit_bytes=64 * 1024 * 1024),
    )(x4, x4, *weights)

    return out[:B][:, None, :]


# speedup vs baseline: 1.1049x; 1.0276x over previous
import functools

import jax
import jax.numpy as jnp
from jax import lax
from jax.experimental import pallas as pl
from jax.experimental.pallas import tpu as pltpu

_F32 = jnp.float32
_BF16 = jnp.bfloat16


def _dtc_body(xa_ref, xb_ref, cw_ref, cb_ref,
              w1ih_ref, w1hh_ref, b1_ref, w2ih_ref, b2_ref, whh2_ref,
              out_ref,
              xg1_scr, act_scr, h1_scr, xg2t_scr, hf_scr, hb_scr,
              h_scr, c_scr,
              *, TC, NP, H, L2, K, Cout, G):
    gid = pl.program_id(0)
    ci = pl.program_id(1)
    NC = pl.num_programs(1) - 1
    w1ih = w1ih_ref[...]
    b1 = b1_ref[...]

    def _conv_act(lo_ref, hi_ref):
        xa = lo_ref[0]
        xb = hi_ref[0][:, :, :K - 1]
        HP = TC // 2 * NP
        for hh in range(2):
            if hh == 0:
                xsl = xa[:, :, :TC + K - 1]
            else:
                xsl = jnp.concatenate([xa[:, :, TC:], xb], axis=2)
            xh = jnp.transpose(xsl, (2, 0, 1))
            xf = xh.reshape((TC + K - 1) * NP, xh.shape[2])
            y = jnp.dot(xf[:TC * NP], cw_ref[0],
                        preferred_element_type=_F32)
            for k in range(1, K):
                y = y + jnp.dot(xf[k * NP:(TC + k) * NP], cw_ref[k],
                                preferred_element_type=_F32)
            y3 = y.reshape(TC // 2, 2 * NP, Cout)
            a = jnp.maximum(y3[:, :NP, :], y3[:, NP:, :]).reshape(HP, Cout)
            a = a + cb_ref[...]
            act_scr[pl.ds(hh * HP, HP), :] = jnp.where(a >= 0.0, a, 0.01 * a)

    _conv_act(xa_ref, xb_ref)

    @pl.when(ci == 0)
    def _init():
        h_scr[...] = jnp.zeros_like(h_scr)
        c_scr[...] = jnp.zeros_like(c_scr)
        xg1_scr[...] = jnp.dot(act_scr[...], w1ih,
                               preferred_element_type=_F32) + b1

    w1hh = w1hh_ref[...]
    NCH = 2
    HNP = NP // NCH

    def _cell(r0, h, c):
        g = xg1_scr[pl.ds(r0, HNP), :] + jnp.dot(
            h, w1hh, preferred_element_type=_F32)
        th = jnp.tanh(g[:, :3 * H])
        tg = jnp.tanh(g[:, 3 * H:])
        c = 0.5 * ((c + tg) + (th[:, H:2 * H] * c + th[:, :H] * tg))
        tc = jnp.tanh(c)
        h = (tc + th[:, 2 * H:3 * H] * tc).astype(_BF16)
        h1_scr[pl.ds(r0, HNP), :] = h
        return h, c

    @pl.when(ci > 0)
    def _recur():
        UN = next(u for u in (16, 8, 4, 2, 1) if TC % u == 0)

        def l1_block(tb, carry):
            hs, cs = carry
            base = pl.multiple_of(tb * UN * NP, NP)
            for u in range(UN):
                r0 = base + u * NP
                new = [_cell(r0 + q * HNP, hs[q], cs[q]) for q in range(NCH)]
                hs = tuple(n[0] for n in new)
                cs = tuple(n[1] for n in new)
                xg1_scr[pl.ds(r0, NP), :] = jnp.dot(
                    act_scr[pl.ds(r0, NP), :], w1ih,
                    preferred_element_type=_F32) + b1
            return hs, cs

        carry0 = (tuple(h_scr[q * HNP:(q + 1) * HNP, :] for q in range(NCH)),
                  tuple(c_scr[q * HNP:(q + 1) * HNP, :] for q in range(NCH)))
        hs, cs = lax.fori_loop(0, TC // UN, l1_block, carry0)
        for q in range(NCH):
            h_scr[q * HNP:(q + 1) * HNP, :] = hs[q]
            c_scr[q * HNP:(q + 1) * HNP, :] = cs[q]

        g2 = jnp.dot(h1_scr[...], w2ih_ref[...],
                     preferred_element_type=_F32) + b2_ref[...]
        g2t = jnp.transpose(g2.reshape(TC, NP, 8),
                            (0, 2, 1)).reshape(TC * 8, NP)
        xg2t_scr[gid,
                 pl.ds(pl.multiple_of((ci - 1) * TC * 8, 8), TC * 8), :] = g2t

    @pl.when((gid == G - 1) & (ci == NC))
    def _lstm2():
        whh2 = whh2_ref[...]
        fmask = (lax.broadcasted_iota(jnp.int32, (8, 1), 0) & 1) == 0

        def _cell2(gq, s, rb, h2, c2):
            row_f = xg2t_scr[gq, pl.ds(pl.multiple_of(s * 8, 8), 8), :]
            row_b = xg2t_scr[gq, pl.ds(pl.multiple_of(rb * 8, 8), 8), :]
            g = jnp.where(fmask, row_f, row_b) + jnp.concatenate(
                [h2, h2, h2, h2], axis=0) * whh2
            sg = 0.5 * jnp.tanh(g[0:6, :]) + 0.5
            gg = jnp.tanh(g[6:8, :])
            c2 = sg[2:4, :] * c2 + sg[0:2, :] * gg
            h2 = sg[4:6, :] * jnp.tanh(c2)
            hf_scr[gq, pl.ds(s, 1), :] = h2[0:1, :]
            hb_scr[gq, pl.ds(rb, 1), :] = h2[1:2, :]
            return h2, c2

        def l2_step(s, carry):
            rb = L2 - 1 - s
            hs2, cs2 = carry
            new = [_cell2(gq, s, rb, hs2[gq], cs2[gq]) for gq in range(G)]
            return tuple(n[0] for n in new), tuple(n[1] for n in new)

        zero2 = jnp.zeros((2, NP), _F32)
        lax.fori_loop(0, L2, l2_step,
                      ((zero2,) * G, (zero2,) * G), unroll=16)
        for gq in range(G):
            out_ref[pl.ds(gq * NP, NP), :] = jnp.transpose(
                hf_scr[gq] + hb_scr[gq])


def kernel(conv_w, conv_b, l1_w_ih, l1_w_hh, l1_b_ih, l1_b_hh,
           l2f_w_ih, l2f_w_hh, l2f_b_ih, l2f_b_hh,
           l2b_w_ih, l2b_w_hh, l2b_b_ih, l2b_b_hh, x):
    Cout, Cin, K = conv_w.shape
    H = l1_w_hh.shape[1]
    B, _, L = x.shape
    L1 = L - K + 1
    L2 = (L1 - 2) // 2 + 1

    NP = 128
    G = pl.cdiv(B, NP)
    B_pad = G * NP
    TC = next(t for t in (64, 32, 16, 8, 4, 2, 1) if L2 % t == 0)
    NC = L2 // TC
    NBL = pl.cdiv(L, 2 * TC)

    x4 = jnp.pad(x.astype(_F32), ((0, B_pad - B), (0, 0), (0, 0)))
    x4 = x4.reshape(G, NP, Cin, L)

    cw = jnp.transpose(conv_w.astype(_F32), (2, 1, 0))
    cb = conv_b.reshape(1, Cout).astype(_F32)

    perm1 = jnp.concatenate([jnp.arange(0, 2 * H),
                             jnp.arange(3 * H, 4 * H),
                             jnp.arange(2 * H, 3 * H)])
    sc1 = jnp.concatenate([jnp.full((3 * H,), 0.5, _F32),
                           jnp.ones((H,), _F32)])[None, :]
    w1ih = (l1_w_ih[perm1, :].T * sc1).astype(_F32)
    w1hh = (l1_w_hh[perm1, :].T * (0.5 * sc1)).astype(_BF16)
    b1 = ((l1_b_ih + l1_b_hh)[perm1].reshape(1, 4 * H) * sc1).astype(_F32)

    perm2 = jnp.array([0, 1, 3, 2])
    w2f = l2f_w_ih[perm2, :].T.astype(_F32)
    w2b = l2b_w_ih[perm2, :].T.astype(_F32)
    sc2 = jnp.concatenate([jnp.full((6,), 0.5, _F32),
                           jnp.ones((2,), _F32)])
    w2ih = (jnp.stack([w2f, w2b], axis=2).reshape(H, 8)
            * sc2[None, :]).astype(_F32)
    b2f = (l2f_b_ih + l2f_b_hh)[perm2]
    b2b = (l2b_b_ih + l2b_b_hh)[perm2]
    b2 = (jnp.stack([b2f, b2b], axis=1).reshape(1, 8)
          * sc2[None, :]).astype(_F32)
    whh2 = (jnp.stack([l2f_w_hh[perm2, 0], l2b_w_hh[perm2, 0]],
                      axis=1).reshape(8, 1) * sc2[:, None]).astype(_F32)

    weights = (cw, cb, w1ih, w1hh, b1, w2ih, b2, whh2)

    def full_spec(a):
        nd = a.ndim
        return pl.BlockSpec(a.shape, lambda g, ci, nd=nd: (0,) * nd)

    def xspec(off):
        return pl.BlockSpec(
            (1, NP, Cin, 2 * TC),
            lambda g, ci, off=off: (g, 0, 0, jnp.minimum(ci + off, NBL - 1)))

    body = functools.partial(_dtc_body, TC=TC, NP=NP, H=H, L2=L2, K=K,
                             Cout=Cout, G=G)

    out = pl.pallas_call(
        body,
        out_shape=jax.ShapeDtypeStruct((B_pad, L2), _F32),
        grid_spec=pltpu.PrefetchScalarGridSpec(
            num_scalar_prefetch=0,
            grid=(G, NC + 1),
            in_specs=[xspec(0), xspec(1)]
                     + [full_spec(a) for a in weights],
            out_specs=pl.BlockSpec((B_pad, L2), lambda g, ci: (0, 0)),
            scratch_shapes=[
                pltpu.VMEM((TC * NP, 4 * H), _F32),
                pltpu.VMEM((TC * NP, Cout), _F32),
                pltpu.VMEM((TC * NP, H), _BF16),
                pltpu.VMEM((G, L2 * 8, NP), _F32),
                pltpu.VMEM((G, L2, NP), _F32),
                pltpu.VMEM((G, L2, NP), _F32),
                pltpu.VMEM((NP, H), _BF16),
                pltpu.VMEM((NP, H), _F32),
            ]),
        compiler_params=pltpu.CompilerParams(
            dimension_semantics=("arbitrary", "arbitrary"),
            vmem_limit_bytes=64 * 1024 * 1024),
    )(x4, x4, *weights)

    return out[:B][:, None, :]


# speedup vs baseline: 1.1123x; 1.0066x over previous
import functools

import jax
import jax.numpy as jnp
from jax import lax
from jax.experimental import pallas as pl
from jax.experimental.pallas import tpu as pltpu

_F32 = jnp.float32
_BF16 = jnp.bfloat16


def _dtc_body(xa_ref, xb_ref, cw_ref, cb_ref,
              w1ih_ref, w1hh_ref, b1_ref, w2ih_ref, b2_ref, whh2_ref,
              out_ref,
              xg1_scr, act_scr, h1_scr, xg2t_scr, hf_scr, hb_scr,
              h_scr, c_scr,
              *, TC, NP, H, L2, K, Cout, G):
    gid = pl.program_id(0)
    ci = pl.program_id(1)
    NC = pl.num_programs(1) - 1
    w1ih = w1ih_ref[...]
    b1 = b1_ref[...]

    def _conv_act(lo_ref, hi_ref):
        xa = lo_ref[0]
        xb = hi_ref[0][:, :, :K - 1]
        HP = TC // 2 * NP
        for hh in range(2):
            if hh == 0:
                xsl = xa[:, :, :TC + K - 1]
            else:
                xsl = jnp.concatenate([xa[:, :, TC:], xb], axis=2)
            xh = jnp.transpose(xsl, (2, 0, 1))
            xf = xh.reshape((TC + K - 1) * NP, xh.shape[2])
            y = jnp.dot(xf[:TC * NP], cw_ref[0],
                        preferred_element_type=_F32)
            for k in range(1, K):
                y = y + jnp.dot(xf[k * NP:(TC + k) * NP], cw_ref[k],
                                preferred_element_type=_F32)
            y3 = y.reshape(TC // 2, 2 * NP, Cout)
            a = jnp.maximum(y3[:, :NP, :], y3[:, NP:, :]).reshape(HP, Cout)
            a = a + cb_ref[...]
            act_scr[pl.ds(hh * HP, HP), :] = jnp.where(a >= 0.0, a, 0.01 * a)

    _conv_act(xa_ref, xb_ref)

    @pl.when(ci == 0)
    def _init():
        h_scr[...] = jnp.zeros_like(h_scr)
        c_scr[...] = jnp.zeros_like(c_scr)
        xg1_scr[...] = jnp.dot(act_scr[...], w1ih,
                               preferred_element_type=_F32) + b1

    w1hh = w1hh_ref[...]
    NCH = 2
    HNP = NP // NCH

    def _cell(r0, h, c):
        g = xg1_scr[pl.ds(r0, HNP), :] + jnp.dot(
            h, w1hh, preferred_element_type=_F32)
        th = jnp.tanh(g[:, :3 * H])
        tg = jnp.tanh(g[:, 3 * H:])
        c = 0.5 * ((c + tg) + (th[:, H:2 * H] * c + th[:, :H] * tg))
        tc = jnp.tanh(c)
        h = (tc + th[:, 2 * H:3 * H] * tc).astype(_BF16)
        h1_scr[pl.ds(r0, HNP), :] = h
        return h, c

    @pl.when(ci > 0)
    def _recur():
        UN = next(u for u in (32, 16, 8, 4, 2, 1) if TC % u == 0)

        def l1_block(tb, carry):
            hs, cs = carry
            base = pl.multiple_of(tb * UN * NP, NP)
            for u in range(UN):
                r0 = base + u * NP
                new = [_cell(r0 + q * HNP, hs[q], cs[q]) for q in range(NCH)]
                hs = tuple(n[0] for n in new)
                cs = tuple(n[1] for n in new)
                xg1_scr[pl.ds(r0, NP), :] = jnp.dot(
                    act_scr[pl.ds(r0, NP), :], w1ih,
                    preferred_element_type=_F32) + b1
            return hs, cs

        carry0 = (tuple(h_scr[q * HNP:(q + 1) * HNP, :] for q in range(NCH)),
                  tuple(c_scr[q * HNP:(q + 1) * HNP, :] for q in range(NCH)))
        hs, cs = lax.fori_loop(0, TC // UN, l1_block, carry0)
        for q in range(NCH):
            h_scr[q * HNP:(q + 1) * HNP, :] = hs[q]
            c_scr[q * HNP:(q + 1) * HNP, :] = cs[q]

        g2 = jnp.dot(h1_scr[...], w2ih_ref[...],
                     preferred_element_type=_F32) + b2_ref[...]
        g2t = jnp.transpose(g2.reshape(TC, NP, 8),
                            (0, 2, 1)).reshape(TC * 8, NP)
        xg2t_scr[gid,
                 pl.ds(pl.multiple_of((ci - 1) * TC * 8, 8), TC * 8), :] = g2t

    @pl.when((gid == G - 1) & (ci == NC))
    def _lstm2():
        whh2 = whh2_ref[...]
        fmask = (lax.broadcasted_iota(jnp.int32, (8, 1), 0) & 1) == 0

        def _cell2(gq, s, rb, h2, c2):
            row_f = xg2t_scr[gq, pl.ds(pl.multiple_of(s * 8, 8), 8), :]
            row_b = xg2t_scr[gq, pl.ds(pl.multiple_of(rb * 8, 8), 8), :]
            g = jnp.where(fmask, row_f, row_b) + jnp.concatenate(
                [h2, h2, h2, h2], axis=0) * whh2
            sg = 0.5 * jnp.tanh(g[0:6, :]) + 0.5
            gg = jnp.tanh(g[6:8, :])
            c2 = sg[2:4, :] * c2 + sg[0:2, :] * gg
            h2 = sg[4:6, :] * jnp.tanh(c2)
            hf_scr[gq, pl.ds(s, 1), :] = h2[0:1, :]
            hb_scr[gq, pl.ds(rb, 1), :] = h2[1:2, :]
            return h2, c2

        def l2_step(s, carry):
            rb = L2 - 1 - s
            hs2, cs2 = carry
            new = [_cell2(gq, s, rb, hs2[gq], cs2[gq]) for gq in range(G)]
            return tuple(n[0] for n in new), tuple(n[1] for n in new)

        zero2 = jnp.zeros((2, NP), _F32)
        lax.fori_loop(0, L2, l2_step,
                      ((zero2,) * G, (zero2,) * G), unroll=32)
        for gq in range(G):
            out_ref[pl.ds(gq * NP, NP), :] = jnp.transpose(
                hf_scr[gq] + hb_scr[gq])


def kernel(conv_w, conv_b, l1_w_ih, l1_w_hh, l1_b_ih, l1_b_hh,
           l2f_w_ih, l2f_w_hh, l2f_b_ih, l2f_b_hh,
           l2b_w_ih, l2b_w_hh, l2b_b_ih, l2b_b_hh, x):
    Cout, Cin, K = conv_w.shape
    H = l1_w_hh.shape[1]
    B, _, L = x.shape
    L1 = L - K + 1
    L2 = (L1 - 2) // 2 + 1

    NP = 128
    G = pl.cdiv(B, NP)
    B_pad = G * NP
    TC = next(t for t in (64, 32, 16, 8, 4, 2, 1) if L2 % t == 0)
    NC = L2 // TC
    NBL = pl.cdiv(L, 2 * TC)

    x4 = jnp.pad(x.astype(_F32), ((0, B_pad - B), (0, 0), (0, 0)))
    x4 = x4.reshape(G, NP, Cin, L)

    cw = jnp.transpose(conv_w.astype(_F32), (2, 1, 0))
    cb = conv_b.reshape(1, Cout).astype(_F32)

    perm1 = jnp.concatenate([jnp.arange(0, 2 * H),
                             jnp.arange(3 * H, 4 * H),
                             jnp.arange(2 * H, 3 * H)])
    sc1 = jnp.concatenate([jnp.full((3 * H,), 0.5, _F32),
                           jnp.ones((H,), _F32)])[None, :]
    w1ih = (l1_w_ih[perm1, :].T * sc1).astype(_F32)
    w1hh = (l1_w_hh[perm1, :].T * (0.5 * sc1)).astype(_BF16)
    b1 = ((l1_b_ih + l1_b_hh)[perm1].reshape(1, 4 * H) * sc1).astype(_F32)

    perm2 = jnp.array([0, 1, 3, 2])
    w2f = l2f_w_ih[perm2, :].T.astype(_F32)
    w2b = l2b_w_ih[perm2, :].T.astype(_F32)
    sc2 = jnp.concatenate([jnp.full((6,), 0.5, _F32),
                           jnp.ones((2,), _F32)])
    w2ih = (jnp.stack([w2f, w2b], axis=2).reshape(H, 8)
            * sc2[None, :]).astype(_F32)
    b2f = (l2f_b_ih + l2f_b_hh)[perm2]
    b2b = (l2b_b_ih + l2b_b_hh)[perm2]
    b2 = (jnp.stack([b2f, b2b], axis=1).reshape(1, 8)
          * sc2[None, :]).astype(_F32)
    whh2 = (jnp.stack([l2f_w_hh[perm2, 0], l2b_w_hh[perm2, 0]],
                      axis=1).reshape(8, 1) * sc2[:, None]).astype(_F32)

    weights = (cw, cb, w1ih, w1hh, b1, w2ih, b2, whh2)

    def full_spec(a):
        nd = a.ndim
        return pl.BlockSpec(a.shape, lambda g, ci, nd=nd: (0,) * nd)

    def xspec(off):
        return pl.BlockSpec(
            (1, NP, Cin, 2 * TC),
            lambda g, ci, off=off: (g, 0, 0, jnp.minimum(ci + off, NBL - 1)))

    body = functools.partial(_dtc_body, TC=TC, NP=NP, H=H, L2=L2, K=K,
                             Cout=Cout, G=G)

    out = pl.pallas_call(
        body,
        out_shape=jax.ShapeDtypeStruct((B_pad, L2), _F32),
        grid_spec=pltpu.PrefetchScalarGridSpec(
            num_scalar_prefetch=0,
            grid=(G, NC + 1),
            in_specs=[xspec(0), xspec(1)]
                     + [full_spec(a) for a in weights],
            out_specs=pl.BlockSpec((B_pad, L2), lambda g, ci: (0, 0)),
            scratch_shapes=[
                pltpu.VMEM((TC * NP, 4 * H), _F32),
                pltpu.VMEM((TC * NP, Cout), _F32),
                pltpu.VMEM((TC * NP, H), _BF16),
                pltpu.VMEM((G, L2 * 8, NP), _F32),
                pltpu.VMEM((G, L2, NP), _F32),
                pltpu.VMEM((G, L2, NP), _F32),
                pltpu.VMEM((NP, H), _BF16),
                pltpu.VMEM((NP, H), _F32),
            ]),
        compiler_params=pltpu.CompilerParams(
            dimension_semantics=("arbitrary", "arbitrary"),
            vmem_limit_bytes=64 * 1024 * 1024),
    )(x4, x4, *weights)

    return out[:B][:, None, :]


# speedup vs baseline: 1.1367x; 1.0219x over previous
import functools

import jax
import jax.numpy as jnp
from jax import lax
from jax.experimental import pallas as pl
from jax.experimental.pallas import tpu as pltpu

_F32 = jnp.float32
_BF16 = jnp.bfloat16


def _dtc_body(xa_ref, xb_ref, cw_ref, cb_ref,
              w1ih_ref, w1hh_ref, b1_ref, w2ih_ref, b2_ref, whh2_ref,
              out_ref,
              xg1_scr, act_scr, h1_scr, xg2t_scr, hf_scr, hb_scr,
              h_scr, c_scr,
              *, TC, NP, H, L2, K, Cout, G):
    gid = pl.program_id(0)
    ci = pl.program_id(1)
    NC = pl.num_programs(1) - 1
    w1ih = w1ih_ref[...]
    b1 = b1_ref[...]

    def _conv_act(lo_ref, hi_ref):
        xa = lo_ref[0]
        xb = hi_ref[0][:, :, :K - 1]
        HP = TC // 2 * NP
        for hh in range(2):
            if hh == 0:
                xsl = xa[:, :, :TC + K - 1]
            else:
                xsl = jnp.concatenate([xa[:, :, TC:], xb], axis=2)
            xh = jnp.transpose(xsl, (2, 0, 1))
            xf = xh.reshape((TC + K - 1) * NP, xh.shape[2])
            y = jnp.dot(xf[:TC * NP], cw_ref[0],
                        preferred_element_type=_F32)
            for k in range(1, K):
                y = y + jnp.dot(xf[k * NP:(TC + k) * NP], cw_ref[k],
                                preferred_element_type=_F32)
            y3 = y.reshape(TC // 2, 2 * NP, Cout)
            a = jnp.maximum(y3[:, :NP, :], y3[:, NP:, :]).reshape(HP, Cout)
            a = a + cb_ref[...]
            act_scr[pl.ds(hh * HP, HP), :] = jnp.where(a >= 0.0, a, 0.01 * a)

    _conv_act(xa_ref, xb_ref)

    @pl.when(ci == 0)
    def _init():
        h_scr[...] = jnp.zeros_like(h_scr)
        c_scr[...] = jnp.zeros_like(c_scr)
        xg1_scr[...] = jnp.dot(act_scr[...], w1ih,
                               preferred_element_type=_F32) + b1

    w1hh = w1hh_ref[...]
    NCH = 2
    HNP = NP // NCH

    def _cell(r0, h, c):
        g = xg1_scr[pl.ds(r0, HNP), :] + jnp.dot(
            h, w1hh, preferred_element_type=_F32)
        th = jnp.tanh(g[:, :3 * H])
        tg = jnp.tanh(g[:, 3 * H:])
        c = 0.5 * ((c + tg) + (th[:, H:2 * H] * c + th[:, :H] * tg))
        tc = jnp.tanh(c)
        h = (tc + th[:, 2 * H:3 * H] * tc).astype(_BF16)
        h1_scr[pl.ds(r0, HNP), :] = h
        return h, c

    @pl.when(ci > 0)
    def _recur():
        UN = next(u for u in (64, 32, 16, 8, 4, 2, 1) if TC % u == 0)

        def l1_block(tb, carry):
            hs, cs = carry
            base = pl.multiple_of(tb * UN * NP, NP)
            for u in range(UN):
                r0 = base + u * NP
                new = [_cell(r0 + q * HNP, hs[q], cs[q]) for q in range(NCH)]
                hs = tuple(n[0] for n in new)
                cs = tuple(n[1] for n in new)
                xg1_scr[pl.ds(r0, NP), :] = jnp.dot(
                    act_scr[pl.ds(r0, NP), :], w1ih,
                    preferred_element_type=_F32) + b1
            return hs, cs

        carry0 = (tuple(h_scr[q * HNP:(q + 1) * HNP, :] for q in range(NCH)),
                  tuple(c_scr[q * HNP:(q + 1) * HNP, :] for q in range(NCH)))
        hs, cs = lax.fori_loop(0, TC // UN, l1_block, carry0)
        for q in range(NCH):
            h_scr[q * HNP:(q + 1) * HNP, :] = hs[q]
            c_scr[q * HNP:(q + 1) * HNP, :] = cs[q]

        g2 = jnp.dot(h1_scr[...], w2ih_ref[...],
                     preferred_element_type=_F32) + b2_ref[...]
        g2t = jnp.transpose(g2.reshape(TC, NP, 8),
                            (0, 2, 1)).reshape(TC * 8, NP)
        xg2t_scr[gid,
                 pl.ds(pl.multiple_of((ci - 1) * TC * 8, 8), TC * 8), :] = g2t

    @pl.when((gid == G - 1) & (ci == NC))
    def _lstm2():
        whh2 = whh2_ref[...]
        fmask = (lax.broadcasted_iota(jnp.int32, (8, 1), 0) & 1) == 0

        def _cell2(gq, s, rb, h2, c2):
            row_f = xg2t_scr[gq, pl.ds(pl.multiple_of(s * 8, 8), 8), :]
            row_b = xg2t_scr[gq, pl.ds(pl.multiple_of(rb * 8, 8), 8), :]
            g = jnp.where(fmask, row_f, row_b) + jnp.concatenate(
                [h2, h2, h2, h2], axis=0) * whh2
            sg = 0.5 * jnp.tanh(g[0:6, :]) + 0.5
            gg = jnp.tanh(g[6:8, :])
            c2 = sg[2:4, :] * c2 + sg[0:2, :] * gg
            h2 = sg[4:6, :] * jnp.tanh(c2)
            hf_scr[gq, pl.ds(s, 1), :] = h2[0:1, :]
            hb_scr[gq, pl.ds(rb, 1), :] = h2[1:2, :]
            return h2, c2

        def l2_step(s, carry):
            rb = L2 - 1 - s
            hs2, cs2 = carry
            new = [_cell2(gq, s, rb, hs2[gq], cs2[gq]) for gq in range(G)]
            return tuple(n[0] for n in new), tuple(n[1] for n in new)

        zero2 = jnp.zeros((2, NP), _F32)
        lax.fori_loop(0, L2, l2_step,
                      ((zero2,) * G, (zero2,) * G), unroll=64)
        for gq in range(G):
            out_ref[pl.ds(gq * NP, NP), :] = jnp.transpose(
                hf_scr[gq] + hb_scr[gq])


def kernel(conv_w, conv_b, l1_w_ih, l1_w_hh, l1_b_ih, l1_b_hh,
           l2f_w_ih, l2f_w_hh, l2f_b_ih, l2f_b_hh,
           l2b_w_ih, l2b_w_hh, l2b_b_ih, l2b_b_hh, x):
    Cout, Cin, K = conv_w.shape
    H = l1_w_hh.shape[1]
    B, _, L = x.shape
    L1 = L - K + 1
    L2 = (L1 - 2) // 2 + 1

    NP = 128
    G = pl.cdiv(B, NP)
    B_pad = G * NP
    TC = next(t for t in (64, 32, 16, 8, 4, 2, 1) if L2 % t == 0)
    NC = L2 // TC
    NBL = pl.cdiv(L, 2 * TC)

    x4 = jnp.pad(x.astype(_F32), ((0, B_pad - B), (0, 0), (0, 0)))
    x4 = x4.reshape(G, NP, Cin, L)

    cw = jnp.transpose(conv_w.astype(_F32), (2, 1, 0))
    cb = conv_b.reshape(1, Cout).astype(_F32)

    perm1 = jnp.concatenate([jnp.arange(0, 2 * H),
                             jnp.arange(3 * H, 4 * H),
                             jnp.arange(2 * H, 3 * H)])
    sc1 = jnp.concatenate([jnp.full((3 * H,), 0.5, _F32),
                           jnp.ones((H,), _F32)])[None, :]
    w1ih = (l1_w_ih[perm1, :].T * sc1).astype(_F32)
    w1hh = (l1_w_hh[perm1, :].T * (0.5 * sc1)).astype(_BF16)
    b1 = ((l1_b_ih + l1_b_hh)[perm1].reshape(1, 4 * H) * sc1).astype(_F32)

    perm2 = jnp.array([0, 1, 3, 2])
    w2f = l2f_w_ih[perm2, :].T.astype(_F32)
    w2b = l2b_w_ih[perm2, :].T.astype(_F32)
    sc2 = jnp.concatenate([jnp.full((6,), 0.5, _F32),
                           jnp.ones((2,), _F32)])
    w2ih = (jnp.stack([w2f, w2b], axis=2).reshape(H, 8)
            * sc2[None, :]).astype(_F32)
    b2f = (l2f_b_ih + l2f_b_hh)[perm2]
    b2b = (l2b_b_ih + l2b_b_hh)[perm2]
    b2 = (jnp.stack([b2f, b2b], axis=1).reshape(1, 8)
          * sc2[None, :]).astype(_F32)
    whh2 = (jnp.stack([l2f_w_hh[perm2, 0], l2b_w_hh[perm2, 0]],
                      axis=1).reshape(8, 1) * sc2[:, None]).astype(_F32)

    weights = (cw, cb, w1ih, w1hh, b1, w2ih, b2, whh2)

    def full_spec(a):
        nd = a.ndim
        return pl.BlockSpec(a.shape, lambda g, ci, nd=nd: (0,) * nd)

    def xspec(off):
        return pl.BlockSpec(
            (1, NP, Cin, 2 * TC),
            lambda g, ci, off=off: (g, 0, 0, jnp.minimum(ci + off, NBL - 1)))

    body = functools.partial(_dtc_body, TC=TC, NP=NP, H=H, L2=L2, K=K,
                             Cout=Cout, G=G)

    out = pl.pallas_call(
        body,
        out_shape=jax.ShapeDtypeStruct((B_pad, L2), _F32),
        grid_spec=pltpu.PrefetchScalarGridSpec(
            num_scalar_prefetch=0,
            grid=(G, NC + 1),
            in_specs=[xspec(0), xspec(1)]
                     + [full_spec(a) for a in weights],
            out_specs=pl.BlockSpec((B_pad, L2), lambda g, ci: (0, 0)),
            scratch_shapes=[
                pltpu.VMEM((TC * NP, 4 * H), _F32),
                pltpu.VMEM((TC * NP, Cout), _F32),
                pltpu.VMEM((TC * NP, H), _BF16),
                pltpu.VMEM((G, L2 * 8, NP), _F32),
                pltpu.VMEM((G, L2, NP), _F32),
                pltpu.VMEM((G, L2, NP), _F32),
                pltpu.VMEM((NP, H), _BF16),
                pltpu.VMEM((NP, H), _F32),
            ]),
        compiler_params=pltpu.CompilerParams(
            dimension_semantics=("arbitrary", "arbitrary"),
            vmem_limit_bytes=64 * 1024 * 1024),
    )(x4, x4, *weights)

    return out[:B][:, None, :]


# speedup vs baseline: 1.1384x; 1.0015x over previous
import functools

import jax
import jax.numpy as jnp
from jax import lax
from jax.experimental import pallas as pl
from jax.experimental.pallas import tpu as pltpu

_F32 = jnp.float32
_BF16 = jnp.bfloat16


def _dtc_body(xa_ref, xb_ref, cw_ref, cb_ref,
              w1ih_ref, w1hh_ref, b1_ref, w2ih_ref, b2_ref, whh2_ref,
              out_ref,
              xg1_scr, act_scr, h1_scr, xg2t_scr, hf_scr, hb_scr,
              h_scr, c_scr,
              *, TC, NP, H, L2, K, Cout, G):
    gid = pl.program_id(0)
    ci = pl.program_id(1)
    NC = pl.num_programs(1) - 1
    w1ih = w1ih_ref[...]
    b1 = b1_ref[...]

    def _conv_act(lo_ref, hi_ref):
        xa = lo_ref[0]
        xb = hi_ref[0][:, :, :K - 1]
        HP = TC // 2 * NP
        for hh in range(2):
            if hh == 0:
                xsl = xa[:, :, :TC + K - 1]
            else:
                xsl = jnp.concatenate([xa[:, :, TC:], xb], axis=2)
            xh = jnp.transpose(xsl, (2, 0, 1))
            xf = xh.reshape((TC + K - 1) * NP, xh.shape[2])
            y = jnp.dot(xf[:TC * NP], cw_ref[0],
                        preferred_element_type=_F32)
            for k in range(1, K):
                y = y + jnp.dot(xf[k * NP:(TC + k) * NP], cw_ref[k],
                                preferred_element_type=_F32)
            y3 = y.reshape(TC // 2, 2 * NP, Cout)
            a = jnp.maximum(y3[:, :NP, :], y3[:, NP:, :]).reshape(HP, Cout)
            a = a + cb_ref[...]
            act_scr[pl.ds(hh * HP, HP), :] = jnp.where(a >= 0.0, a, 0.01 * a)

    _conv_act(xa_ref, xb_ref)

    @pl.when(ci == 0)
    def _init():
        h_scr[...] = jnp.zeros_like(h_scr)
        c_scr[...] = jnp.zeros_like(c_scr)
        xg1_scr[...] = jnp.dot(act_scr[...], w1ih,
                               preferred_element_type=_F32) + b1

    w1hh = w1hh_ref[...]
    NCH = 2
    HNP = NP // NCH

    def _cell(r0, h, c):
        g = xg1_scr[pl.ds(r0, HNP), :] + jnp.dot(
            h, w1hh, preferred_element_type=_F32)
        th = jnp.tanh(g[:, :3 * H])
        tg = jnp.tanh(g[:, 3 * H:])
        c = 0.5 * ((c + tg) + (th[:, H:2 * H] * c + th[:, :H] * tg))
        tc = jnp.tanh(c)
        h = (tc + th[:, 2 * H:3 * H] * tc).astype(_BF16)
        h1_scr[pl.ds(r0, HNP), :] = h
        return h, c

    @pl.when(ci > 0)
    def _recur():
        UN = next(u for u in (64, 32, 16, 8, 4, 2, 1) if TC % u == 0)

        def l1_block(tb, carry):
            hs, cs = carry
            base = pl.multiple_of(tb * UN * NP, NP)
            for u in range(UN):
                r0 = base + u * NP
                new = [_cell(r0 + q * HNP, hs[q], cs[q]) for q in range(NCH)]
                hs = tuple(n[0] for n in new)
                cs = tuple(n[1] for n in new)
                xg1_scr[pl.ds(r0, NP), :] = jnp.dot(
                    act_scr[pl.ds(r0, NP), :], w1ih,
                    preferred_element_type=_F32) + b1
            return hs, cs

        carry0 = (tuple(h_scr[q * HNP:(q + 1) * HNP, :] for q in range(NCH)),
                  tuple(c_scr[q * HNP:(q + 1) * HNP, :] for q in range(NCH)))
        hs, cs = lax.fori_loop(0, TC // UN, l1_block, carry0)
        for q in range(NCH):
            h_scr[q * HNP:(q + 1) * HNP, :] = hs[q]
            c_scr[q * HNP:(q + 1) * HNP, :] = cs[q]

        g2 = jnp.dot(h1_scr[...], w2ih_ref[...],
                     preferred_element_type=_F32) + b2_ref[...]
        g2t = jnp.transpose(g2.reshape(TC, NP, 8),
                            (0, 2, 1)).reshape(TC * 8, NP)
        xg2t_scr[gid,
                 pl.ds(pl.multiple_of((ci - 1) * TC * 8, 8), TC * 8), :] = g2t

    @pl.when((gid == G - 1) & (ci == NC))
    def _lstm2():
        whh2 = whh2_ref[...]
        fmask = (lax.broadcasted_iota(jnp.int32, (8, 1), 0) & 1) == 0

        def _cell2(gq, s, rb, h2, c2):
            row_f = xg2t_scr[gq, pl.ds(pl.multiple_of(s * 8, 8), 8), :]
            row_b = xg2t_scr[gq, pl.ds(pl.multiple_of(rb * 8, 8), 8), :]
            g = jnp.where(fmask, row_f, row_b) + jnp.concatenate(
                [h2, h2, h2, h2], axis=0) * whh2
            sg = 0.5 * jnp.tanh(g[0:6, :]) + 0.5
            gg = jnp.tanh(g[6:8, :])
            c2 = sg[2:4, :] * c2 + sg[0:2, :] * gg
            h2 = sg[4:6, :] * jnp.tanh(c2)
            hf_scr[gq, pl.ds(s, 1), :] = h2[0:1, :]
            hb_scr[gq, pl.ds(rb, 1), :] = h2[1:2, :]
            return h2, c2

        def l2_step(s, carry):
            rb = L2 - 1 - s
            hs2, cs2 = carry
            new = [_cell2(gq, s, rb, hs2[gq], cs2[gq]) for gq in range(G)]
            return tuple(n[0] for n in new), tuple(n[1] for n in new)

        zero2 = jnp.zeros((2, NP), _F32)
        lax.fori_loop(0, L2, l2_step,
                      ((zero2,) * G, (zero2,) * G), unroll=128)
        for gq in range(G):
            out_ref[pl.ds(gq * NP, NP), :] = jnp.transpose(
                hf_scr[gq] + hb_scr[gq])


def kernel(conv_w, conv_b, l1_w_ih, l1_w_hh, l1_b_ih, l1_b_hh,
           l2f_w_ih, l2f_w_hh, l2f_b_ih, l2f_b_hh,
           l2b_w_ih, l2b_w_hh, l2b_b_ih, l2b_b_hh, x):
    Cout, Cin, K = conv_w.shape
    H = l1_w_hh.shape[1]
    B, _, L = x.shape
    L1 = L - K + 1
    L2 = (L1 - 2) // 2 + 1

    NP = 128
    G = pl.cdiv(B, NP)
    B_pad = G * NP
    TC = next(t for t in (64, 32, 16, 8, 4, 2, 1) if L2 % t == 0)
    NC = L2 // TC
    NBL = pl.cdiv(L, 2 * TC)

    x4 = jnp.pad(x.astype(_F32), ((0, B_pad - B), (0, 0), (0, 0)))
    x4 = x4.reshape(G, NP, Cin, L)

    cw = jnp.transpose(conv_w.astype(_F32), (2, 1, 0))
    cb = conv_b.reshape(1, Cout).astype(_F32)

    perm1 = jnp.concatenate([jnp.arange(0, 2 * H),
                             jnp.arange(3 * H, 4 * H),
                             jnp.arange(2 * H, 3 * H)])
    sc1 = jnp.concatenate([jnp.full((3 * H,), 0.5, _F32),
                           jnp.ones((H,), _F32)])[None, :]
    w1ih = (l1_w_ih[perm1, :].T * sc1).astype(_F32)
    w1hh = (l1_w_hh[perm1, :].T * (0.5 * sc1)).astype(_BF16)
    b1 = ((l1_b_ih + l1_b_hh)[perm1].reshape(1, 4 * H) * sc1).astype(_F32)

    perm2 = jnp.array([0, 1, 3, 2])
    w2f = l2f_w_ih[perm2, :].T.astype(_F32)
    w2b = l2b_w_ih[perm2, :].T.astype(_F32)
    sc2 = jnp.concatenate([jnp.full((6,), 0.5, _F32),
                           jnp.ones((2,), _F32)])
    w2ih = (jnp.stack([w2f, w2b], axis=2).reshape(H, 8)
            * sc2[None, :]).astype(_F32)
    b2f = (l2f_b_ih + l2f_b_hh)[perm2]
    b2b = (l2b_b_ih + l2b_b_hh)[perm2]
    b2 = (jnp.stack([b2f, b2b], axis=1).reshape(1, 8)
          * sc2[None, :]).astype(_F32)
    whh2 = (jnp.stack([l2f_w_hh[perm2, 0], l2b_w_hh[perm2, 0]],
                      axis=1).reshape(8, 1) * sc2[:, None]).astype(_F32)

    weights = (cw, cb, w1ih, w1hh, b1, w2ih, b2, whh2)

    def full_spec(a):
        nd = a.ndim
        return pl.BlockSpec(a.shape, lambda g, ci, nd=nd: (0,) * nd)

    def xspec(off):
        return pl.BlockSpec(
            (1, NP, Cin, 2 * TC),
            lambda g, ci, off=off: (g, 0, 0, jnp.minimum(ci + off, NBL - 1)))

    body = functools.partial(_dtc_body, TC=TC, NP=NP, H=H, L2=L2, K=K,
                             Cout=Cout, G=G)

    out = pl.pallas_call(
        body,
        out_shape=jax.ShapeDtypeStruct((B_pad, L2), _F32),
        grid_spec=pltpu.PrefetchScalarGridSpec(
            num_scalar_prefetch=0,
            grid=(G, NC + 1),
            in_specs=[xspec(0), xspec(1)]
                     + [full_spec(a) for a in weights],
            out_specs=pl.BlockSpec((B_pad, L2), lambda g, ci: (0, 0)),
            scratch_shapes=[
                pltpu.VMEM((TC * NP, 4 * H), _F32),
                pltpu.VMEM((TC * NP, Cout), _F32),
                pltpu.VMEM((TC * NP, H), _BF16),
                pltpu.VMEM((G, L2 * 8, NP), _F32),
                pltpu.VMEM((G, L2, NP), _F32),
                pltpu.VMEM((G, L2, NP), _F32),
                pltpu.VMEM((NP, H), _BF16),
                pltpu.VMEM((NP, H), _F32),
            ]),
        compiler_params=pltpu.CompilerParams(
            dimension_semantics=("arbitrary", "arbitrary"),
            vmem_limit_bytes=64 * 1024 * 1024),
    )(x4, x4, *weights)

    return out[:B][:, None, :]


# speedup vs baseline: 1.2385x; 1.0879x over previous
import functools

import jax
import jax.numpy as jnp
from jax import lax
from jax.experimental import pallas as pl
from jax.experimental.pallas import tpu as pltpu

_F32 = jnp.float32
_BF16 = jnp.bfloat16


def _dtc_body(xa_ref, xb_ref, cw_ref, cb_ref,
              w1ih_ref, w1hh_ref, b1_ref, w2ih_ref, b2_ref, whh2_ref,
              out_ref,
              xg1_scr, act_scr, h1_scr, xg2t_scr, hf_scr, hb_scr,
              h_scr, c_scr,
              *, TC, NP, H, L2, K, Cout, G):
    gid = pl.program_id(0)
    ci = pl.program_id(1)
    NC = pl.num_programs(1) - 1
    w1ih = w1ih_ref[...]
    b1 = b1_ref[...]

    def _conv_act(lo_ref, hi_ref):
        xa = lo_ref[0].astype(_BF16)
        xb = hi_ref[0][:, :, :K - 1].astype(_BF16)
        HP = TC // 2 * NP
        for hh in range(2):
            if hh == 0:
                xsl = xa[:, :, :TC + K - 1]
            else:
                xsl = jnp.concatenate([xa[:, :, TC:], xb], axis=2)
            xh = jnp.transpose(xsl, (2, 0, 1))
            xf = xh.reshape((TC + K - 1) * NP, xh.shape[2])
            y = jnp.dot(xf[:TC * NP], cw_ref[0],
                        preferred_element_type=_F32)
            for k in range(1, K):
                y = y + jnp.dot(xf[k * NP:(TC + k) * NP], cw_ref[k],
                                preferred_element_type=_F32)
            y3 = y.reshape(TC // 2, 2 * NP, Cout)
            a = jnp.maximum(y3[:, :NP, :], y3[:, NP:, :]).reshape(HP, Cout)
            a = a + cb_ref[...]
            act_scr[pl.ds(hh * HP, HP), :] = jnp.where(a >= 0.0, a, 0.01 * a)

    _conv_act(xa_ref, xb_ref)

    @pl.when(ci == 0)
    def _init():
        h_scr[...] = jnp.zeros_like(h_scr)
        c_scr[...] = jnp.zeros_like(c_scr)
        xg1_scr[...] = jnp.dot(act_scr[...], w1ih,
                               preferred_element_type=_F32) + b1

    w1hh = w1hh_ref[...]
    NCH = 2
    HNP = NP // NCH

    def _cell(r0, h, c):
        g = xg1_scr[pl.ds(r0, HNP), :] + jnp.dot(
            h, w1hh, preferred_element_type=_F32)
        th = jnp.tanh(g[:, :3 * H])
        tg = jnp.tanh(g[:, 3 * H:])
        c = 0.5 * ((c + tg) + (th[:, H:2 * H] * c + th[:, :H] * tg))
        tc = jnp.tanh(c)
        h = (tc + th[:, 2 * H:3 * H] * tc).astype(_BF16)
        h1_scr[pl.ds(r0, HNP), :] = h
        return h, c

    @pl.when(ci > 0)
    def _recur():
        UN = next(u for u in (64, 32, 16, 8, 4, 2, 1) if TC % u == 0)

        def l1_block(tb, carry):
            hs, cs = carry
            base = pl.multiple_of(tb * UN * NP, NP)
            for u in range(UN):
                r0 = base + u * NP
                new = [_cell(r0 + q * HNP, hs[q], cs[q]) for q in range(NCH)]
                hs = tuple(n[0] for n in new)
                cs = tuple(n[1] for n in new)
                xg1_scr[pl.ds(r0, NP), :] = jnp.dot(
                    act_scr[pl.ds(r0, NP), :], w1ih,
                    preferred_element_type=_F32) + b1
            return hs, cs

        carry0 = (tuple(h_scr[q * HNP:(q + 1) * HNP, :] for q in range(NCH)),
                  tuple(c_scr[q * HNP:(q + 1) * HNP, :] for q in range(NCH)))
        hs, cs = lax.fori_loop(0, TC // UN, l1_block, carry0)
        for q in range(NCH):
            h_scr[q * HNP:(q + 1) * HNP, :] = hs[q]
            c_scr[q * HNP:(q + 1) * HNP, :] = cs[q]

        g2 = jnp.dot(h1_scr[...], w2ih_ref[...],
                     preferred_element_type=_F32) + b2_ref[...]
        g2t = jnp.transpose(g2.reshape(TC, NP, 8),
                            (0, 2, 1)).reshape(TC * 8, NP)
        xg2t_scr[gid,
                 pl.ds(pl.multiple_of((ci - 1) * TC * 8, 8), TC * 8), :] = g2t

    @pl.when((gid == G - 1) & (ci == NC))
    def _lstm2():
        whh2 = whh2_ref[...]
        fmask = (lax.broadcasted_iota(jnp.int32, (8, 1), 0) & 1) == 0

        def _cell2(gq, s, rb, h2, c2):
            row_f = xg2t_scr[gq, pl.ds(pl.multiple_of(s * 8, 8), 8), :]
            row_b = xg2t_scr[gq, pl.ds(pl.multiple_of(rb * 8, 8), 8), :]
            g = jnp.where(fmask, row_f, row_b) + jnp.concatenate(
                [h2, h2, h2, h2], axis=0) * whh2
            sg = 0.5 * jnp.tanh(g[0:6, :]) + 0.5
            gg = jnp.tanh(g[6:8, :])
            c2 = sg[2:4, :] * c2 + sg[0:2, :] * gg
            h2 = sg[4:6, :] * jnp.tanh(c2)
            hf_scr[gq, pl.ds(s, 1), :] = h2[0:1, :]
            hb_scr[gq, pl.ds(rb, 1), :] = h2[1:2, :]
            return h2, c2

        def l2_step(s, carry):
            rb = L2 - 1 - s
            hs2, cs2 = carry
            new = [_cell2(gq, s, rb, hs2[gq], cs2[gq]) for gq in range(G)]
            return tuple(n[0] for n in new), tuple(n[1] for n in new)

        zero2 = jnp.zeros((2, NP), _F32)
        lax.fori_loop(0, L2, l2_step,
                      ((zero2,) * G, (zero2,) * G), unroll=128)
        for gq in range(G):
            out_ref[pl.ds(gq * NP, NP), :] = jnp.transpose(
                hf_scr[gq] + hb_scr[gq])


def kernel(conv_w, conv_b, l1_w_ih, l1_w_hh, l1_b_ih, l1_b_hh,
           l2f_w_ih, l2f_w_hh, l2f_b_ih, l2f_b_hh,
           l2b_w_ih, l2b_w_hh, l2b_b_ih, l2b_b_hh, x):
    Cout, Cin, K = conv_w.shape
    H = l1_w_hh.shape[1]
    B, _, L = x.shape
    L1 = L - K + 1
    L2 = (L1 - 2) // 2 + 1

    NP = 128
    G = pl.cdiv(B, NP)
    B_pad = G * NP
    TC = next(t for t in (64, 32, 16, 8, 4, 2, 1) if L2 % t == 0)
    NC = L2 // TC
    NBL = pl.cdiv(L, 2 * TC)

    x4 = jnp.pad(x.astype(_F32), ((0, B_pad - B), (0, 0), (0, 0)))
    x4 = x4.reshape(G, NP, Cin, L)

    cw = jnp.transpose(conv_w.astype(_BF16), (2, 1, 0))
    cb = conv_b.reshape(1, Cout).astype(_F32)

    perm1 = jnp.concatenate([jnp.arange(0, 2 * H),
                             jnp.arange(3 * H, 4 * H),
                             jnp.arange(2 * H, 3 * H)])
    sc1 = jnp.concatenate([jnp.full((3 * H,), 0.5, _F32),
                           jnp.ones((H,), _F32)])[None, :]
    w1ih = (l1_w_ih[perm1, :].T * sc1).astype(_F32)
    w1hh = (l1_w_hh[perm1, :].T * (0.5 * sc1)).astype(_BF16)
    b1 = ((l1_b_ih + l1_b_hh)[perm1].reshape(1, 4 * H) * sc1).astype(_F32)

    perm2 = jnp.array([0, 1, 3, 2])
    w2f = l2f_w_ih[perm2, :].T.astype(_F32)
    w2b = l2b_w_ih[perm2, :].T.astype(_F32)
    sc2 = jnp.concatenate([jnp.full((6,), 0.5, _F32),
                           jnp.ones((2,), _F32)])
    w2ih = (jnp.stack([w2f, w2b], axis=2).reshape(H, 8)
            * sc2[None, :]).astype(_F32)
    b2f = (l2f_b_ih + l2f_b_hh)[perm2]
    b2b = (l2b_b_ih + l2b_b_hh)[perm2]
    b2 = (jnp.stack([b2f, b2b], axis=1).reshape(1, 8)
          * sc2[None, :]).astype(_F32)
    whh2 = (jnp.stack([l2f_w_hh[perm2, 0], l2b_w_hh[perm2, 0]],
                      axis=1).reshape(8, 1) * sc2[:, None]).astype(_F32)

    weights = (cw, cb, w1ih, w1hh, b1, w2ih, b2, whh2)

    def full_spec(a):
        nd = a.ndim
        return pl.BlockSpec(a.shape, lambda g, ci, nd=nd: (0,) * nd)

    def xspec(off):
        return pl.BlockSpec(
            (1, NP, Cin, 2 * TC),
            lambda g, ci, off=off: (g, 0, 0, jnp.minimum(ci + off, NBL - 1)))

    body = functools.partial(_dtc_body, TC=TC, NP=NP, H=H, L2=L2, K=K,
                             Cout=Cout, G=G)

    out = pl.pallas_call(
        body,
        out_shape=jax.ShapeDtypeStruct((B_pad, L2), _F32),
        grid_spec=pltpu.PrefetchScalarGridSpec(
            num_scalar_prefetch=0,
            grid=(G, NC + 1),
            in_specs=[xspec(0), xspec(1)]
                     + [full_spec(a) for a in weights],
            out_specs=pl.BlockSpec((B_pad, L2), lambda g, ci: (0, 0)),
            scratch_shapes=[
                pltpu.VMEM((TC * NP, 4 * H), _F32),
                pltpu.VMEM((TC * NP, Cout), _F32),
                pltpu.VMEM((TC * NP, H), _BF16),
                pltpu.VMEM((G, L2 * 8, NP), _F32),
                pltpu.VMEM((G, L2, NP), _F32),
                pltpu.VMEM((G, L2, NP), _F32),
                pltpu.VMEM((NP, H), _BF16),
                pltpu.VMEM((NP, H), _F32),
            ]),
        compiler_params=pltpu.CompilerParams(
            dimension_semantics=("arbitrary", "arbitrary"),
            vmem_limit_bytes=64 * 1024 * 1024),
    )(x4, x4, *weights)

    return out[:B][:, None, :]


# speedup vs baseline: 1.2386x; 1.0001x over previous
import functools

import jax
import jax.numpy as jnp
from jax import lax
from jax.experimental import pallas as pl
from jax.experimental.pallas import tpu as pltpu

_F32 = jnp.float32
_BF16 = jnp.bfloat16


def _dtc_body(xa_ref, xb_ref, cw_ref, cb_ref,
              w1ih_ref, w1hh_ref, b1_ref, w2ih_ref, b2_ref, whh2_ref,
              out_ref,
              xg1_scr, act_scr, h1_scr, xg2t_scr, hf_scr, hb_scr,
              h_scr, c_scr,
              *, TC, NP, H, L2, K, Cout, G):
    gid = pl.program_id(0)
    ci = pl.program_id(1)
    NC = pl.num_programs(1) - 1
    w1ih = w1ih_ref[...]
    b1 = b1_ref[...]

    def _conv_act(lo_ref, hi_ref):
        xa = lo_ref[0].astype(_BF16)
        xb = hi_ref[0][:, :, :K - 1].astype(_BF16)
        HP = TC // 2 * NP
        for hh in range(2):
            if hh == 0:
                xsl = xa[:, :, :TC + K - 1]
            else:
                xsl = jnp.concatenate([xa[:, :, TC:], xb], axis=2)
            xh = jnp.transpose(xsl, (2, 0, 1))
            xf = xh.reshape((TC + K - 1) * NP, xh.shape[2])
            y = jnp.dot(xf[:TC * NP], cw_ref[0],
                        preferred_element_type=_F32)
            for k in range(1, K):
                y = y + jnp.dot(xf[k * NP:(TC + k) * NP], cw_ref[k],
                                preferred_element_type=_F32)
            y3 = y.reshape(TC // 2, 2 * NP, Cout)
            a = jnp.maximum(y3[:, :NP, :], y3[:, NP:, :]).reshape(HP, Cout)
            a = a + cb_ref[...]
            act_scr[pl.ds(hh * HP, HP), :] = jnp.where(
                a >= 0.0, a, 0.01 * a).astype(_BF16)

    _conv_act(xa_ref, xb_ref)

    @pl.when(ci == 0)
    def _init():
        h_scr[...] = jnp.zeros_like(h_scr)
        c_scr[...] = jnp.zeros_like(c_scr)
        xg1_scr[...] = jnp.dot(act_scr[...], w1ih,
                               preferred_element_type=_F32) + b1

    w1hh = w1hh_ref[...]
    NCH = 2
    HNP = NP // NCH

    def _cell(r0, h, c):
        g = xg1_scr[pl.ds(r0, HNP), :] + jnp.dot(
            h, w1hh, preferred_element_type=_F32)
        th = jnp.tanh(g[:, :3 * H])
        tg = jnp.tanh(g[:, 3 * H:])
        c = 0.5 * ((c + tg) + (th[:, H:2 * H] * c + th[:, :H] * tg))
        tc = jnp.tanh(c)
        h = (tc + th[:, 2 * H:3 * H] * tc).astype(_BF16)
        h1_scr[pl.ds(r0, HNP), :] = h
        return h, c

    @pl.when(ci > 0)
    def _recur():
        UN = next(u for u in (64, 32, 16, 8, 4, 2, 1) if TC % u == 0)

        def l1_block(tb, carry):
            hs, cs = carry
            base = pl.multiple_of(tb * UN * NP, NP)
            for u in range(UN):
                r0 = base + u * NP
                new = [_cell(r0 + q * HNP, hs[q], cs[q]) for q in range(NCH)]
                hs = tuple(n[0] for n in new)
                cs = tuple(n[1] for n in new)
                xg1_scr[pl.ds(r0, NP), :] = jnp.dot(
                    act_scr[pl.ds(r0, NP), :], w1ih,
                    preferred_element_type=_F32) + b1
            return hs, cs

        carry0 = (tuple(h_scr[q * HNP:(q + 1) * HNP, :] for q in range(NCH)),
                  tuple(c_scr[q * HNP:(q + 1) * HNP, :] for q in range(NCH)))
        hs, cs = lax.fori_loop(0, TC // UN, l1_block, carry0)
        for q in range(NCH):
            h_scr[q * HNP:(q + 1) * HNP, :] = hs[q]
            c_scr[q * HNP:(q + 1) * HNP, :] = cs[q]

        g2 = jnp.dot(h1_scr[...], w2ih_ref[...],
                     preferred_element_type=_F32) + b2_ref[...]
        g2t = jnp.transpose(g2.reshape(TC, NP, 8),
                            (0, 2, 1)).reshape(TC * 8, NP)
        xg2t_scr[gid,
                 pl.ds(pl.multiple_of((ci - 1) * TC * 8, 8), TC * 8), :] = g2t

    @pl.when((gid == G - 1) & (ci == NC))
    def _lstm2():
        whh2 = whh2_ref[...]
        fmask = (lax.broadcasted_iota(jnp.int32, (8, 1), 0) & 1) == 0

        def _cell2(gq, s, rb, h2, c2):
            row_f = xg2t_scr[gq, pl.ds(pl.multiple_of(s * 8, 8), 8), :]
            row_b = xg2t_scr[gq, pl.ds(pl.multiple_of(rb * 8, 8), 8), :]
            g = jnp.where(fmask, row_f, row_b) + jnp.concatenate(
                [h2, h2, h2, h2], axis=0) * whh2
            sg = 0.5 * jnp.tanh(g[0:6, :]) + 0.5
            gg = jnp.tanh(g[6:8, :])
            c2 = sg[2:4, :] * c2 + sg[0:2, :] * gg
            h2 = sg[4:6, :] * jnp.tanh(c2)
            hf_scr[gq, pl.ds(s, 1), :] = h2[0:1, :]
            hb_scr[gq, pl.ds(rb, 1), :] = h2[1:2, :]
            return h2, c2

        def l2_step(s, carry):
            rb = L2 - 1 - s
            hs2, cs2 = carry
            new = [_cell2(gq, s, rb, hs2[gq], cs2[gq]) for gq in range(G)]
            return tuple(n[0] for n in new), tuple(n[1] for n in new)

        zero2 = jnp.zeros((2, NP), _F32)
        lax.fori_loop(0, L2, l2_step,
                      ((zero2,) * G, (zero2,) * G), unroll=128)
        for gq in range(G):
            out_ref[pl.ds(gq * NP, NP), :] = jnp.transpose(
                hf_scr[gq] + hb_scr[gq])


def kernel(conv_w, conv_b, l1_w_ih, l1_w_hh, l1_b_ih, l1_b_hh,
           l2f_w_ih, l2f_w_hh, l2f_b_ih, l2f_b_hh,
           l2b_w_ih, l2b_w_hh, l2b_b_ih, l2b_b_hh, x):
    Cout, Cin, K = conv_w.shape
    H = l1_w_hh.shape[1]
    B, _, L = x.shape
    L1 = L - K + 1
    L2 = (L1 - 2) // 2 + 1

    NP = 128
    G = pl.cdiv(B, NP)
    B_pad = G * NP
    TC = next(t for t in (64, 32, 16, 8, 4, 2, 1) if L2 % t == 0)
    NC = L2 // TC
    NBL = pl.cdiv(L, 2 * TC)

    x4 = jnp.pad(x.astype(_F32), ((0, B_pad - B), (0, 0), (0, 0)))
    x4 = x4.reshape(G, NP, Cin, L)

    cw = jnp.transpose(conv_w.astype(_BF16), (2, 1, 0))
    cb = conv_b.reshape(1, Cout).astype(_F32)

    perm1 = jnp.concatenate([jnp.arange(0, 2 * H),
                             jnp.arange(3 * H, 4 * H),
                             jnp.arange(2 * H, 3 * H)])
    sc1 = jnp.concatenate([jnp.full((3 * H,), 0.5, _F32),
                           jnp.ones((H,), _F32)])[None, :]
    w1ih = (l1_w_ih[perm1, :].T * sc1).astype(_BF16)
    w1hh = (l1_w_hh[perm1, :].T * (0.5 * sc1)).astype(_BF16)
    b1 = ((l1_b_ih + l1_b_hh)[perm1].reshape(1, 4 * H) * sc1).astype(_F32)

    perm2 = jnp.array([0, 1, 3, 2])
    w2f = l2f_w_ih[perm2, :].T.astype(_F32)
    w2b = l2b_w_ih[perm2, :].T.astype(_F32)
    sc2 = jnp.concatenate([jnp.full((6,), 0.5, _F32),
                           jnp.ones((2,), _F32)])
    w2ih = (jnp.stack([w2f, w2b], axis=2).reshape(H, 8)
            * sc2[None, :]).astype(_F32)
    b2f = (l2f_b_ih + l2f_b_hh)[perm2]
    b2b = (l2b_b_ih + l2b_b_hh)[perm2]
    b2 = (jnp.stack([b2f, b2b], axis=1).reshape(1, 8)
          * sc2[None, :]).astype(_F32)
    whh2 = (jnp.stack([l2f_w_hh[perm2, 0], l2b_w_hh[perm2, 0]],
                      axis=1).reshape(8, 1) * sc2[:, None]).astype(_F32)

    weights = (cw, cb, w1ih, w1hh, b1, w2ih, b2, whh2)

    def full_spec(a):
        nd = a.ndim
        return pl.BlockSpec(a.shape, lambda g, ci, nd=nd: (0,) * nd)

    def xspec(off):
        return pl.BlockSpec(
            (1, NP, Cin, 2 * TC),
            lambda g, ci, off=off: (g, 0, 0, jnp.minimum(ci + off, NBL - 1)))

    body = functools.partial(_dtc_body, TC=TC, NP=NP, H=H, L2=L2, K=K,
                             Cout=Cout, G=G)

    out = pl.pallas_call(
        body,
        out_shape=jax.ShapeDtypeStruct((B_pad, L2), _F32),
        grid_spec=pltpu.PrefetchScalarGridSpec(
            num_scalar_prefetch=0,
            grid=(G, NC + 1),
            in_specs=[xspec(0), xspec(1)]
                     + [full_spec(a) for a in weights],
            out_specs=pl.BlockSpec((B_pad, L2), lambda g, ci: (0, 0)),
            scratch_shapes=[
                pltpu.VMEM((TC * NP, 4 * H), _F32),
                pltpu.VMEM((TC * NP, Cout), _BF16),
                pltpu.VMEM((TC * NP, H), _BF16),
                pltpu.VMEM((G, L2 * 8, NP), _F32),
                pltpu.VMEM((G, L2, NP), _F32),
                pltpu.VMEM((G, L2, NP), _F32),
                pltpu.VMEM((NP, H), _BF16),
                pltpu.VMEM((NP, H), _F32),
            ]),
        compiler_params=pltpu.CompilerParams(
            dimension_semantics=("arbitrary", "arbitrary"),
            vmem_limit_bytes=64 * 1024 * 1024),
    )(x4, x4, *weights)

    return out[:B][:, None, :]


# speedup vs baseline: 1.2984x; 1.0482x over previous
import functools

import jax
import jax.numpy as jnp
from jax import lax
from jax.experimental import pallas as pl
from jax.experimental.pallas import tpu as pltpu

_F32 = jnp.float32
_BF16 = jnp.bfloat16


def _dtc_body(xa_ref, xb_ref, cw_ref, cb_ref,
              w1ih_ref, w1hh_ref, b1_ref, w2ih_ref, b2_ref, whh2_ref,
              out_ref,
              xg1_scr, act_scr, h1_scr, xg2t_scr, hf_scr, hb_scr,
              h_scr, c_scr,
              *, TC, NP, H, L2, K, Cout, G):
    gid = pl.program_id(0)
    ci = pl.program_id(1)
    NC = pl.num_programs(1) - 1
    w1ih = w1ih_ref[...]
    b1 = b1_ref[...]

    def _conv_act(lo_ref, hi_ref):
        xa = lo_ref[0].astype(_BF16)
        xb = hi_ref[0][:, :, :K - 1].astype(_BF16)
        HP = TC // 2 * NP
        for hh in range(2):
            if hh == 0:
                xsl = xa[:, :, :TC + K - 1]
            else:
                xsl = jnp.concatenate([xa[:, :, TC:], xb], axis=2)
            xh = jnp.transpose(xsl, (2, 0, 1))
            xf = xh.reshape((TC + K - 1) * NP, xh.shape[2])
            y = jnp.dot(xf[:TC * NP], cw_ref[0],
                        preferred_element_type=_F32)
            for k in range(1, K):
                y = y + jnp.dot(xf[k * NP:(TC + k) * NP], cw_ref[k],
                                preferred_element_type=_F32)
            y3 = y.reshape(TC // 2, 2 * NP, Cout)
            a = jnp.maximum(y3[:, :NP, :], y3[:, NP:, :]).reshape(HP, Cout)
            a = a + cb_ref[...]
            act_scr[pl.ds(hh * HP, HP), :] = jnp.where(
                a >= 0.0, a, 0.01 * a).astype(_BF16)

    _conv_act(xa_ref, xb_ref)

    @pl.when(ci == 0)
    def _init():
        h_scr[...] = jnp.zeros_like(h_scr)
        c_scr[...] = jnp.zeros_like(c_scr)
        xg1_scr[...] = jnp.dot(act_scr[...], w1ih,
                               preferred_element_type=_F32) + b1

    w1hh = w1hh_ref[...]
    NCH = 4
    HNP = NP // NCH

    def _cell(r0, h, c):
        g = xg1_scr[pl.ds(r0, HNP), :] + jnp.dot(
            h, w1hh, preferred_element_type=_F32)
        th = jnp.tanh(g[:, :3 * H])
        tg = jnp.tanh(g[:, 3 * H:])
        c = 0.5 * ((c + tg) + (th[:, H:2 * H] * c + th[:, :H] * tg))
        tc = jnp.tanh(c)
        h = (tc + th[:, 2 * H:3 * H] * tc).astype(_BF16)
        h1_scr[pl.ds(r0, HNP), :] = h
        return h, c

    @pl.when(ci > 0)
    def _recur():
        UN = next(u for u in (64, 32, 16, 8, 4, 2, 1) if TC % u == 0)

        def l1_block(tb, carry):
            hs, cs = carry
            base = pl.multiple_of(tb * UN * NP, NP)
            for u in range(UN):
                r0 = base + u * NP
                new = [_cell(r0 + q * HNP, hs[q], cs[q]) for q in range(NCH)]
                hs = tuple(n[0] for n in new)
                cs = tuple(n[1] for n in new)
                xg1_scr[pl.ds(r0, NP), :] = jnp.dot(
                    act_scr[pl.ds(r0, NP), :], w1ih,
                    preferred_element_type=_F32) + b1
            return hs, cs

        carry0 = (tuple(h_scr[q * HNP:(q + 1) * HNP, :] for q in range(NCH)),
                  tuple(c_scr[q * HNP:(q + 1) * HNP, :] for q in range(NCH)))
        hs, cs = lax.fori_loop(0, TC // UN, l1_block, carry0)
        for q in range(NCH):
            h_scr[q * HNP:(q + 1) * HNP, :] = hs[q]
            c_scr[q * HNP:(q + 1) * HNP, :] = cs[q]

        g2 = jnp.dot(h1_scr[...], w2ih_ref[...],
                     preferred_element_type=_F32) + b2_ref[...]
        g2t = jnp.transpose(g2.reshape(TC, NP, 8),
                            (0, 2, 1)).reshape(TC * 8, NP)
        xg2t_scr[gid,
                 pl.ds(pl.multiple_of((ci - 1) * TC * 8, 8), TC * 8), :] = g2t

    @pl.when((gid == G - 1) & (ci == NC))
    def _lstm2():
        whh2 = whh2_ref[...]
        fmask = (lax.broadcasted_iota(jnp.int32, (8, 1), 0) & 1) == 0

        def _cell2(gq, s, rb, h2, c2):
            row_f = xg2t_scr[gq, pl.ds(pl.multiple_of(s * 8, 8), 8), :]
            row_b = xg2t_scr[gq, pl.ds(pl.multiple_of(rb * 8, 8), 8), :]
            g = jnp.where(fmask, row_f, row_b) + jnp.concatenate(
                [h2, h2, h2, h2], axis=0) * whh2
            sg = 0.5 * jnp.tanh(g[0:6, :]) + 0.5
            gg = jnp.tanh(g[6:8, :])
            c2 = sg[2:4, :] * c2 + sg[0:2, :] * gg
            h2 = sg[4:6, :] * jnp.tanh(c2)
            hf_scr[gq, pl.ds(s, 1), :] = h2[0:1, :]
            hb_scr[gq, pl.ds(rb, 1), :] = h2[1:2, :]
            return h2, c2

        def l2_step(s, carry):
            rb = L2 - 1 - s
            hs2, cs2 = carry
            new = [_cell2(gq, s, rb, hs2[gq], cs2[gq]) for gq in range(G)]
            return tuple(n[0] for n in new), tuple(n[1] for n in new)

        zero2 = jnp.zeros((2, NP), _F32)
        lax.fori_loop(0, L2, l2_step,
                      ((zero2,) * G, (zero2,) * G), unroll=128)
        for gq in range(G):
            out_ref[pl.ds(gq * NP, NP), :] = jnp.transpose(
                hf_scr[gq] + hb_scr[gq])


def kernel(conv_w, conv_b, l1_w_ih, l1_w_hh, l1_b_ih, l1_b_hh,
           l2f_w_ih, l2f_w_hh, l2f_b_ih, l2f_b_hh,
           l2b_w_ih, l2b_w_hh, l2b_b_ih, l2b_b_hh, x):
    Cout, Cin, K = conv_w.shape
    H = l1_w_hh.shape[1]
    B, _, L = x.shape
    L1 = L - K + 1
    L2 = (L1 - 2) // 2 + 1

    NP = 128
    G = pl.cdiv(B, NP)
    B_pad = G * NP
    TC = next(t for t in (64, 32, 16, 8, 4, 2, 1) if L2 % t == 0)
    NC = L2 // TC
    NBL = pl.cdiv(L, 2 * TC)

    x4 = jnp.pad(x.astype(_F32), ((0, B_pad - B), (0, 0), (0, 0)))
    x4 = x4.reshape(G, NP, Cin, L)

    cw = jnp.transpose(conv_w.astype(_BF16), (2, 1, 0))
    cb = conv_b.reshape(1, Cout).astype(_F32)

    perm1 = jnp.concatenate([jnp.arange(0, 2 * H),
                             jnp.arange(3 * H, 4 * H),
                             jnp.arange(2 * H, 3 * H)])
    sc1 = jnp.concatenate([jnp.full((3 * H,), 0.5, _F32),
                           jnp.ones((H,), _F32)])[None, :]
    w1ih = (l1_w_ih[perm1, :].T * sc1).astype(_BF16)
    w1hh = (l1_w_hh[perm1, :].T * (0.5 * sc1)).astype(_BF16)
    b1 = ((l1_b_ih + l1_b_hh)[perm1].reshape(1, 4 * H) * sc1).astype(_F32)

    perm2 = jnp.array([0, 1, 3, 2])
    w2f = l2f_w_ih[perm2, :].T.astype(_F32)
    w2b = l2b_w_ih[perm2, :].T.astype(_F32)
    sc2 = jnp.concatenate([jnp.full((6,), 0.5, _F32),
                           jnp.ones((2,), _F32)])
    w2ih = (jnp.stack([w2f, w2b], axis=2).reshape(H, 8)
            * sc2[None, :]).astype(_F32)
    b2f = (l2f_b_ih + l2f_b_hh)[perm2]
    b2b = (l2b_b_ih + l2b_b_hh)[perm2]
    b2 = (jnp.stack([b2f, b2b], axis=1).reshape(1, 8)
          * sc2[None, :]).astype(_F32)
    whh2 = (jnp.stack([l2f_w_hh[perm2, 0], l2b_w_hh[perm2, 0]],
                      axis=1).reshape(8, 1) * sc2[:, None]).astype(_F32)

    weights = (cw, cb, w1ih, w1hh, b1, w2ih, b2, whh2)

    def full_spec(a):
        nd = a.ndim
        return pl.BlockSpec(a.shape, lambda g, ci, nd=nd: (0,) * nd)

    def xspec(off):
        return pl.BlockSpec(
            (1, NP, Cin, 2 * TC),
            lambda g, ci, off=off: (g, 0, 0, jnp.minimum(ci + off, NBL - 1)))

    body = functools.partial(_dtc_body, TC=TC, NP=NP, H=H, L2=L2, K=K,
                             Cout=Cout, G=G)

    out = pl.pallas_call(
        body,
        out_shape=jax.ShapeDtypeStruct((B_pad, L2), _F32),
        grid_spec=pltpu.PrefetchScalarGridSpec(
            num_scalar_prefetch=0,
            grid=(G, NC + 1),
            in_specs=[xspec(0), xspec(1)]
                     + [full_spec(a) for a in weights],
            out_specs=pl.BlockSpec((B_pad, L2), lambda g, ci: (0, 0)),
            scratch_shapes=[
                pltpu.VMEM((TC * NP, 4 * H), _F32),
                pltpu.VMEM((TC * NP, Cout), _BF16),
                pltpu.VMEM((TC * NP, H), _BF16),
                pltpu.VMEM((G, L2 * 8, NP), _F32),
                pltpu.VMEM((G, L2, NP), _F32),
                pltpu.VMEM((G, L2, NP), _F32),
                pltpu.VMEM((NP, H), _BF16),
                pltpu.VMEM((NP, H), _F32),
            ]),
        compiler_params=pltpu.CompilerParams(
            dimension_semantics=("arbitrary", "arbitrary"),
            vmem_limit_bytes=64 * 1024 * 1024),
    )(x4, x4, *weights)

    return out[:B][:, None, :]
